# baseline (speedup 1.0000x reference)
import numpy as np
import jax
import jax.numpy as jnp
from jax.experimental import pallas as pl
from jax.experimental.pallas import tpu as pltpu

_EPS = float(np.finfo(np.float32).eps)

_B = 16
_N = 112
_K = 8
_E1 = 3
_DIN = 16
_H = 32
_P = 8
_G = 8
_GRID = _B // _G
_ROWS = _G * _N


def _gather_kernel(nf_ref, emb_ref, out_ref, sem):
    base = pl.program_id(0) * _ROWS

    def issue(r, carry):
        row = base + r
        pltpu.make_async_copy(emb_ref.at[pl.ds(nf_ref[row], 1), :],
                              out_ref.at[pl.ds(row, 1), :], sem).start()
        return carry

    jax.lax.fori_loop(0, _ROWS, issue, 0)
    pltpu.make_async_copy(out_ref.at[pl.ds(base, _ROWS), :],
                          out_ref.at[pl.ds(base, _ROWS), :], sem).wait()


def _fwd_kernel(s_ref, idx_ref, m_ref, w0_ref, b0_ref, wl_ref, bl_ref,
                wa_ref, ba_ref, out_ref):
    S = s_ref[...]
    w0 = w0_ref[...] * (1.0 / _K)
    R = [jnp.dot(S, w0[j * _DIN:(j + 1) * _DIN, :],
                 preferred_element_type=jnp.float32) for j in range(_E1)]

    wro = jnp.concatenate([wl_ref[...], wa_ref[...]], axis=1)
    bro = jnp.concatenate([bl_ref[...], ba_ref[...]], axis=1)

    iota_m = jax.lax.broadcasted_iota(jnp.int32, (_N, _N), 1)
    hs = []
    for g in range(_G):
        idx_g = idx_ref[g * _N:(g + 1) * _N, :]
        acc = None
        for j in range(_E1):
            c = jnp.zeros((_N, _N), jnp.float32)
            for k in range(_K):
                col = k * _E1 + j
                c = c + (idx_g[:, col:col + 1] == iota_m).astype(jnp.float32)
            part = jnp.dot(c, R[j][g * _N:(g + 1) * _N, :],
                           preferred_element_type=jnp.float32)
            acc = part if acc is None else acc + part
        hs.append(acc * m_ref[g * _N:(g + 1) * _N, :])
    h = jnp.concatenate(hs, axis=0)

    h = jnp.maximum(h + b0_ref[...], 0.0)
    norm = jnp.sqrt(jnp.sum(h * h, axis=-1, keepdims=True))
    h = h * pl.reciprocal(norm + _EPS, approx=False)

    y_all = jnp.dot(h, wro, preferred_element_type=jnp.float32) + bro
    att = jax.nn.sigmoid(y_all[:, _P:_P + 1])
    contrib = att * y_all[:, :_P]

    means = [jnp.mean(contrib[g * _N:(g + 1) * _N, :], axis=0, keepdims=True)
             for g in range(_G)]
    out_ref[...] = jnp.concatenate(means, axis=0)


def kernel(embedding, filter_w_0, filter_b_0, filter_w_last, filter_b_last,
           att_w, att_b, node_feat, nn_idx, nonempty_mask):
    gathered = pl.pallas_call(
        _gather_kernel,
        out_shape=jax.ShapeDtypeStruct((_B * _N, _DIN), jnp.float32),
        grid_spec=pltpu.PrefetchScalarGridSpec(
            num_scalar_prefetch=1,
            grid=(_GRID,),
            in_specs=[pl.BlockSpec(memory_space=pl.ANY)],
            out_specs=pl.BlockSpec(memory_space=pl.ANY),
            scratch_shapes=[pltpu.SemaphoreType.DMA],
        ),
        compiler_params=pltpu.CompilerParams(
            dimension_semantics=("parallel",)),
    )(node_feat.reshape(-1), embedding)

    state = gathered
    idx = nn_idx.reshape(_B * _N, _K * _E1)
    nmask = nonempty_mask.reshape(_B * _N, 1)

    return pl.pallas_call(
        _fwd_kernel,
        out_shape=jax.ShapeDtypeStruct((_B, _P), jnp.float32),
        grid=(_GRID,),
        in_specs=[
            pl.BlockSpec((_ROWS, _DIN), lambda i: (i, 0)),
            pl.BlockSpec((_ROWS, _K * _E1), lambda i: (i, 0)),
            pl.BlockSpec((_ROWS, 1), lambda i: (i, 0)),
            pl.BlockSpec((_E1 * _DIN, _H), lambda i: (0, 0)),
            pl.BlockSpec((1, _H), lambda i: (0, 0)),
            pl.BlockSpec((_H, _P), lambda i: (0, 0)),
            pl.BlockSpec((1, _P), lambda i: (0, 0)),
            pl.BlockSpec((_H, 1), lambda i: (0, 0)),
            pl.BlockSpec((1, 1), lambda i: (0, 0)),
        ],
        out_specs=pl.BlockSpec((_G, _P), lambda i: (i, 0)),
        compiler_params=pltpu.CompilerParams(
            dimension_semantics=("parallel",)),
    )(state, idx, nmask, filter_w_0, filter_b_0, filter_w_last,
      filter_b_last, att_w, att_b)


# speedup vs baseline: 3.3673x; 3.3673x over previous
import numpy as np
import jax
import jax.numpy as jnp
from jax.experimental import pallas as pl
from jax.experimental.pallas import tpu as pltpu

_EPS = float(np.finfo(np.float32).eps)

_B = 16
_N = 112
_K = 8
_E1 = 3
_DIN = 16
_H = 32
_P = 8
_G = 8
_GRID = _B // _G
_ROWS = _G * _N


def _fwd_kernel(s_ref, idx_ref, m_ref, w0_ref, b0_ref, wl_ref, bl_ref,
                wa_ref, ba_ref, out_ref):
    S = s_ref[...]
    w0 = w0_ref[...] * (1.0 / _K)
    R = [jnp.dot(S, w0[j * _DIN:(j + 1) * _DIN, :],
                 preferred_element_type=jnp.float32) for j in range(_E1)]

    wro = jnp.concatenate([wl_ref[...], wa_ref[...]], axis=1)
    bro = jnp.concatenate([bl_ref[...], ba_ref[...]], axis=1)

    iota_m = jax.lax.broadcasted_iota(jnp.int32, (_N, _N), 1)
    hs = []
    for g in range(_G):
        idx_g = idx_ref[g * _N:(g + 1) * _N, :]
        acc = None
        for j in range(_E1):
            c = jnp.zeros((_N, _N), jnp.float32)
            for k in range(_K):
                col = k * _E1 + j
                c = c + (idx_g[:, col:col + 1] == iota_m).astype(jnp.float32)
            part = jnp.dot(c, R[j][g * _N:(g + 1) * _N, :],
                           preferred_element_type=jnp.float32)
            acc = part if acc is None else acc + part
        hs.append(acc * m_ref[g * _N:(g + 1) * _N, :])
    h = jnp.concatenate(hs, axis=0)

    h = jnp.maximum(h + b0_ref[...], 0.0)
    norm = jnp.sqrt(jnp.sum(h * h, axis=-1, keepdims=True))
    h = h * pl.reciprocal(norm + _EPS, approx=False)

    y_all = jnp.dot(h, wro, preferred_element_type=jnp.float32) + bro
    att = jax.nn.sigmoid(y_all[:, _P:_P + 1])
    contrib = att * y_all[:, :_P]

    means = [jnp.mean(contrib[g * _N:(g + 1) * _N, :], axis=0, keepdims=True)
             for g in range(_G)]
    out_ref[...] = jnp.concatenate(means, axis=0)


def kernel(embedding, filter_w_0, filter_b_0, filter_w_last, filter_b_last,
           att_w, att_b, node_feat, nn_idx, nonempty_mask):
    state = jnp.take(embedding, node_feat.reshape(-1), axis=0)
    idx = nn_idx.reshape(_B * _N, _K * _E1)
    nmask = nonempty_mask.reshape(_B * _N, 1)

    return pl.pallas_call(
        _fwd_kernel,
        out_shape=jax.ShapeDtypeStruct((_B, _P), jnp.float32),
        grid=(_GRID,),
        in_specs=[
            pl.BlockSpec((_ROWS, _DIN), lambda i: (i, 0)),
            pl.BlockSpec((_ROWS, _K * _E1), lambda i: (i, 0)),
            pl.BlockSpec((_ROWS, 1), lambda i: (i, 0)),
            pl.BlockSpec((_E1 * _DIN, _H), lambda i: (0, 0)),
            pl.BlockSpec((1, _H), lambda i: (0, 0)),
            pl.BlockSpec((_H, _P), lambda i: (0, 0)),
            pl.BlockSpec((1, _P), lambda i: (0, 0)),
            pl.BlockSpec((_H, 1), lambda i: (0, 0)),
            pl.BlockSpec((1, 1), lambda i: (0, 0)),
        ],
        out_specs=pl.BlockSpec((_G, _P), lambda i: (i, 0)),
        compiler_params=pltpu.CompilerParams(
            dimension_semantics=("parallel",)),
    )(state, idx, nmask, filter_w_0, filter_b_0, filter_w_last,
      filter_b_last, att_w, att_b)
